# Initial kernel scaffold; baseline (speedup 1.0000x reference)
import functools

import jax
import jax.numpy as jnp
from jax import lax
from jax.experimental import pallas as pl
from jax.experimental.pallas import tpu as pltpu
from jax.experimental.pallas import tpu_sc as plsc

_NC = 2
_NS = 16
_NW = _NC * _NS
_CH = 128
_NBUF = 4


def _gather_body(idx_hbm, table_hbm, out_hbm, idx_v, rows_v, sem):
    nchunk = idx_hbm.shape[0]
    m = nchunk // _NW
    wid = lax.axis_index("s") * _NC + lax.axis_index("c")
    c0 = wid * m

    pltpu.sync_copy(idx_hbm.at[pl.ds(c0, m)], idx_v)

    def _start(g, b):
        pltpu.async_copy(table_hbm.at[idx_v.at[g]], rows_v.at[b], sem)

    def _wait_one(b):
        pltpu.make_async_copy(table_hbm.at[idx_v.at[0]], rows_v.at[b], sem).wait()

    for b in range(_NBUF):
        _start(b, b)

    def _group(i, _):
        go = i * _NBUF
        for b in range(_NBUF):
            g = go + b
            _wait_one(b)
            pltpu.sync_copy(rows_v.at[b], out_hbm.at[c0 + g])
            nxt = g + _NBUF

            @pl.when(nxt < m)
            def _():
                _start(nxt, b)

        return 0

    lax.fori_loop(0, m // _NBUF, _group, 0)


def _sc_gather(idx2d, table):
    nchunk, ch = idx2d.shape
    v, d = table.shape
    m = nchunk // _NW
    run = pl.kernel(
        _gather_body,
        mesh=plsc.VectorSubcoreMesh(core_axis_name="c", subcore_axis_name="s"),
        out_type=jax.ShapeDtypeStruct((nchunk, ch, d), jnp.float32),
        scratch_types=[
            pltpu.VMEM((m, ch), jnp.int32),
            pltpu.VMEM((_NBUF, ch, d), jnp.float32),
            pltpu.SemaphoreType.DMA,
        ],
    )
    return run(idx2d, table)


def _proj_body(f_ref, wt_ref, b_ref, o_ref):
    o_ref[...] = (
        jnp.dot(f_ref[...], wt_ref[...], preferred_element_type=jnp.float32)
        + b_ref[...]
    )


def _tc_project(feats, wt, b):
    n, d = feats.shape
    blk = 4096
    return pl.pallas_call(
        _proj_body,
        grid=(n // blk,),
        in_specs=[
            pl.BlockSpec((blk, d), lambda i: (i, 0)),
            pl.BlockSpec((d, d), lambda i: (0, 0)),
            pl.BlockSpec((1, d), lambda i: (0, 0)),
        ],
        out_specs=pl.BlockSpec((blk, d), lambda i: (i, 0)),
        out_shape=jax.ShapeDtypeStruct((n, d), jnp.float32),
    )(feats, wt, b.reshape(1, d))


def kernel(image, table, W, b):
    bsz, seq = image.shape
    v, d = table.shape
    n = bsz * seq
    idx2d = image.reshape(n // _CH, _CH).astype(jnp.int32)
    feats3 = _sc_gather(idx2d, table)
    feats = feats3.reshape(n, d)
    out = _tc_project(feats, W.T, b)
    return out.reshape(bsz, seq, d)


# speedup vs baseline: 1.1781x; 1.1781x over previous
import functools

import jax
import jax.numpy as jnp
from jax import lax
from jax.experimental import pallas as pl
from jax.experimental.pallas import tpu as pltpu
from jax.experimental.pallas import tpu_sc as plsc

_NC = 2
_NS = 16
_NW = _NC * _NS
_CH = 128
_NBUF = 4


def _gather_body(idx_hbm, table_hbm, out_hbm, idx_v, rows_v, sem):
    nchunk = idx_hbm.shape[0]
    m = nchunk // _NW
    wid = lax.axis_index("s") * _NC + lax.axis_index("c")
    c0 = wid * m

    pltpu.sync_copy(idx_hbm.at[pl.ds(c0, m)], idx_v)

    def _start(g, b):
        pltpu.async_copy(table_hbm.at[idx_v.at[g]], rows_v.at[b], sem)

    def _wait_one(b):
        pltpu.make_async_copy(table_hbm.at[idx_v.at[0]], rows_v.at[b], sem).wait()

    for b in range(_NBUF):
        _start(b, b)

    def _group(i, _):
        go = i * _NBUF
        for b in range(_NBUF):
            g = go + b
            _wait_one(b)
            pltpu.sync_copy(rows_v.at[b], out_hbm.at[c0 + g])
            nxt = g + _NBUF

            @pl.when(nxt < m)
            def _():
                _start(nxt, b)

        return 0

    lax.fori_loop(0, m // _NBUF, _group, 0)


def _sc_gather(idx2d, table):
    nchunk, ch = idx2d.shape
    v, d = table.shape
    m = nchunk // _NW
    run = pl.kernel(
        _gather_body,
        mesh=plsc.VectorSubcoreMesh(core_axis_name="c", subcore_axis_name="s"),
        out_type=jax.ShapeDtypeStruct((nchunk, ch, d), jnp.float32),
        scratch_types=[
            pltpu.VMEM((m, ch), jnp.int32),
            pltpu.VMEM((_NBUF, ch, d), jnp.float32),
            pltpu.SemaphoreType.DMA,
        ],
        compiler_params=pltpu.CompilerParams(use_tc_tiling_on_sc=False),
    )
    return run(idx2d, table)


def _proj_body(f_ref, wt_ref, b_ref, o_ref):
    o_ref[...] = (
        jnp.dot(f_ref[...], wt_ref[...], preferred_element_type=jnp.float32)
        + b_ref[...]
    )


def _tc_project(feats, wt, b):
    n, d = feats.shape
    blk = 4096
    return pl.pallas_call(
        _proj_body,
        grid=(n // blk,),
        in_specs=[
            pl.BlockSpec((blk, d), lambda i: (i, 0)),
            pl.BlockSpec((d, d), lambda i: (0, 0)),
            pl.BlockSpec((1, d), lambda i: (0, 0)),
        ],
        out_specs=pl.BlockSpec((blk, d), lambda i: (i, 0)),
        out_shape=jax.ShapeDtypeStruct((n, d), jnp.float32),
    )(feats, wt, b.reshape(1, d))


def kernel(image, table, W, b):
    bsz, seq = image.shape
    v, d = table.shape
    n = bsz * seq
    idx2d = image.reshape(n // _CH, _CH).astype(jnp.int32)
    feats3 = _sc_gather(idx2d, table)
    feats = feats3.reshape(n, d)
    out = _tc_project(feats, W.T, b)
    return out.reshape(bsz, seq, d)


# speedup vs baseline: 1.3336x; 1.1320x over previous
import functools

import jax
import jax.numpy as jnp
from jax import lax
from jax.experimental import pallas as pl
from jax.experimental.pallas import tpu as pltpu
from jax.experimental.pallas import tpu_sc as plsc

_NC = 2
_NS = 16
_NW = _NC * _NS
_NBUF = 4


def _gather_body(img_hbm, table_hbm, out_hbm, idx_v, rows_v, sem):
    nrow = img_hbm.shape[0]
    m = nrow // _NW
    wid = lax.axis_index("s") * _NC + lax.axis_index("c")
    r0 = wid * m

    pltpu.sync_copy(img_hbm.at[pl.ds(r0, m)], idx_v)

    def _start(g, b):
        pltpu.async_copy(table_hbm.at[idx_v.at[g]], rows_v.at[b], sem)

    def _wait_one(b):
        pltpu.make_async_copy(table_hbm.at[idx_v.at[0]], rows_v.at[b], sem).wait()

    for b in range(_NBUF):
        _start(b, b)

    def _group(i, _):
        go = i * _NBUF
        for b in range(_NBUF):
            g = go + b
            _wait_one(b)
            pltpu.sync_copy(rows_v.at[b], out_hbm.at[r0 + g])
            nxt = g + _NBUF

            @pl.when(nxt < m)
            def _():
                _start(nxt, b)

        return 0

    lax.fori_loop(0, m // _NBUF, _group, 0)


def _sc_gather(image, table):
    nrow, seq = image.shape
    v, d = table.shape
    m = nrow // _NW
    run = pl.kernel(
        _gather_body,
        mesh=plsc.VectorSubcoreMesh(core_axis_name="c", subcore_axis_name="s"),
        out_type=jax.ShapeDtypeStruct((nrow, seq, d), jnp.float32),
        scratch_types=[
            pltpu.VMEM((m, seq), jnp.int32),
            pltpu.VMEM((_NBUF, seq, d), jnp.float32),
            pltpu.SemaphoreType.DMA,
        ],
        compiler_params=pltpu.CompilerParams(use_tc_tiling_on_sc=False),
    )
    return run(image, table)


def _proj_body(f_ref, wt_ref, b_ref, o_ref):
    o_ref[...] = (
        lax.dot_general(
            f_ref[...], wt_ref[...],
            (((2,), (0,)), ((), ())),
            preferred_element_type=jnp.float32,
        )
        + b_ref[...]
    )


def _tc_project(feats, wt, b):
    nrow, seq, d = feats.shape
    blk = 512
    return pl.pallas_call(
        _proj_body,
        grid=(nrow // blk,),
        in_specs=[
            pl.BlockSpec((blk, seq, d), lambda i: (i, 0, 0)),
            pl.BlockSpec((d, d), lambda i: (0, 0)),
            pl.BlockSpec((1, 1, d), lambda i: (0, 0, 0)),
        ],
        out_specs=pl.BlockSpec((blk, seq, d), lambda i: (i, 0, 0)),
        out_shape=jax.ShapeDtypeStruct((nrow, seq, d), jnp.float32),
    )(feats, wt, b.reshape(1, 1, d))


def kernel(image, table, W, b):
    feats = _sc_gather(image.astype(jnp.int32), table)
    return _tc_project(feats, W.T, b)


# speedup vs baseline: 1.5330x; 1.1495x over previous
import functools

import jax
import jax.numpy as jnp
from jax import lax
from jax.experimental import pallas as pl
from jax.experimental.pallas import tpu as pltpu
from jax.experimental.pallas import tpu_sc as plsc

_NC = 2
_NS = 16
_NW = _NC * _NS
_NBUF = 4
_VBLK = 12800
_NBLK = 512


def _pack_body(l_ref, r_ref, o_ref):
    o_ref[...] = lax.dot_general(
        l_ref[...], r_ref[...],
        (((0,), (0,)), ((), ())),
        preferred_element_type=jnp.float32,
    )


def _tc_pack_table(l2, rperm):
    k, h = l2.shape
    return pl.pallas_call(
        _pack_body,
        grid=(pl.cdiv(h, _VBLK),),
        in_specs=[
            pl.BlockSpec((k, _VBLK), lambda j: (0, j)),
            pl.BlockSpec((k, k), lambda j: (0, 0)),
        ],
        out_specs=pl.BlockSpec((_VBLK, k), lambda j: (j, 0)),
        out_shape=jax.ShapeDtypeStruct((h, k), jnp.float32),
    )(l2, rperm)


def _gather_body(img_hbm, table_hbm, out_hbm, idx_v, rows_v, sem):
    nrow = img_hbm.shape[0]
    m = nrow // _NW
    wid = lax.axis_index("s") * _NC + lax.axis_index("c")
    r0 = wid * m

    pltpu.sync_copy(img_hbm.at[pl.ds(r0, m)], idx_v)

    def _start(g, b):
        pltpu.async_copy(table_hbm.at[idx_v.at[g]], rows_v.at[b], sem)

    def _wait_one(b):
        pltpu.make_async_copy(table_hbm.at[idx_v.at[0]], rows_v.at[b], sem).wait()

    for b in range(_NBUF):
        _start(b, b)

    def _group(i, _):
        go = i * _NBUF
        for b in range(_NBUF):
            g = go + b
            _wait_one(b)
            pltpu.sync_copy(rows_v.at[b], out_hbm.at[r0 + g])
            nxt = g + _NBUF

            @pl.when(nxt < m)
            def _():
                _start(nxt, b)

        return 0

    lax.fori_loop(0, m // _NBUF, _group, 0)


def _sc_gather(image, table):
    nrow, seq = image.shape
    v, d = table.shape
    m = nrow // _NW
    run = pl.kernel(
        _gather_body,
        mesh=plsc.VectorSubcoreMesh(core_axis_name="c", subcore_axis_name="s"),
        out_type=jax.ShapeDtypeStruct((nrow, seq, d), jnp.float32),
        scratch_types=[
            pltpu.VMEM((m, seq), jnp.int32),
            pltpu.VMEM((_NBUF, seq, d), jnp.float32),
            pltpu.SemaphoreType.DMA,
        ],
        compiler_params=pltpu.CompilerParams(use_tc_tiling_on_sc=False),
    )
    return run(image, table)


def _proj_body(f_ref, w_ref, b_ref, o_ref):
    f = f_ref[...]
    w = w_ref[...]
    bias = b_ref[...]
    seq = o_ref.shape[0]
    d = w.shape[0]
    for l in range(seq):
        o_ref[l] = (
            lax.dot_general(
                w, f[:, d * l:d * (l + 1)],
                (((1,), (1,)), ((), ())),
                preferred_element_type=jnp.float32,
            )
            + bias
        )


def _tc_project(feats2, w, bcol, seq):
    nrow, ld = feats2.shape
    d = ld // seq
    return pl.pallas_call(
        _proj_body,
        grid=(nrow // _NBLK,),
        in_specs=[
            pl.BlockSpec((_NBLK, ld), lambda j: (j, 0)),
            pl.BlockSpec((d, d), lambda j: (0, 0)),
            pl.BlockSpec((d, 1), lambda j: (0, 0)),
        ],
        out_specs=pl.BlockSpec((seq, d, _NBLK), lambda j: (0, 0, j)),
        out_shape=jax.ShapeDtypeStruct((seq, d, nrow), jnp.float32),
    )(feats2, w, bcol)


def kernel(image, table, W, b):
    v, d = table.shape
    nrow, seq = image.shape
    half = v // 2

    c = jnp.arange(2 * d)
    kappa = 2 * (c % d) + c // d
    rperm = (jnp.arange(2 * d)[:, None] == kappa[None, :]).astype(jnp.float32)

    l2 = table.T.reshape(2 * d, half)
    q2 = _tc_pack_table(l2, rperm)
    q = q2.reshape(v, d)

    im = image.astype(jnp.int32)
    im2 = jnp.where(im < half, 2 * im, 2 * (im - half) + 1)

    feats = _sc_gather(im2, q)
    feats2 = feats.reshape(nrow, seq * d)

    out_t = _tc_project(feats2, W, b.reshape(d, 1), seq)
    return jnp.transpose(out_t, (2, 0, 1))


# speedup vs baseline: 1.5734x; 1.0263x over previous
import functools

import jax
import jax.numpy as jnp
from jax import lax
from jax.experimental import pallas as pl
from jax.experimental.pallas import tpu as pltpu
from jax.experimental.pallas import tpu_sc as plsc

_NC = 2
_NS = 16
_NW = _NC * _NS
_NBUF = 4
_K = 12800
_NBLK = 512


def _pack_body(t_ref, i_ref, o_ref):
    eye = i_ref[...]
    a = lax.dot_general(
        t_ref[:, :_K], eye, (((0,), (0,)), ((), ())),
        preferred_element_type=jnp.float32,
    )
    bb = lax.dot_general(
        t_ref[:, _K:], eye, (((0,), (0,)), ((), ())),
        preferred_element_type=jnp.float32,
    )
    o_ref[:, 0:64] = a
    o_ref[:, 64:128] = bb


def _tc_pack_table(table_t):
    d, v = table_t.shape
    nblk = pl.cdiv(v, 2 * _K)
    return pl.pallas_call(
        _pack_body,
        grid=(nblk,),
        in_specs=[
            pl.BlockSpec((d, 2 * _K), lambda j: (0, j)),
            pl.BlockSpec((d, d), lambda j: (0, 0)),
        ],
        out_specs=pl.BlockSpec((_K, 2 * d), lambda j: (j, 0)),
        out_shape=jax.ShapeDtypeStruct((nblk * _K, 2 * d), jnp.float32),
    )(table_t, jnp.eye(d, dtype=jnp.float32))


def _gather_body(img_hbm, table_hbm, out_hbm, idx_v, rows_v, sem):
    nrow = img_hbm.shape[0]
    half = nrow // 2
    m = nrow // _NW
    wid = lax.axis_index("s") * _NC + lax.axis_index("c")
    r0 = wid * m
    in_hi = r0 >= half
    m0 = jnp.where(in_hi, r0 - half, r0)
    off = jnp.where(in_hi, 64, 0)

    pltpu.sync_copy(img_hbm.at[pl.ds(r0, m)], idx_v)

    def _start(g, b):
        pltpu.async_copy(table_hbm.at[idx_v.at[g]], rows_v.at[b], sem)

    def _wait_one(b):
        pltpu.make_async_copy(table_hbm.at[idx_v.at[0]], rows_v.at[b], sem).wait()

    for b in range(_NBUF):
        _start(b, b)

    def _group(i, _):
        go = i * _NBUF
        for b in range(_NBUF):
            g = go + b
            _wait_one(b)
            pltpu.sync_copy(rows_v.at[b], out_hbm.at[:, m0 + g, pl.ds(off, 64)])
            nxt = g + _NBUF

            @pl.when(nxt < m)
            def _():
                _start(nxt, b)

        return 0

    lax.fori_loop(0, m // _NBUF, _group, 0)


def _sc_gather(image, table):
    nrow, seq = image.shape
    v, d = table.shape
    m = nrow // _NW
    run = pl.kernel(
        _gather_body,
        mesh=plsc.VectorSubcoreMesh(core_axis_name="c", subcore_axis_name="s"),
        out_type=jax.ShapeDtypeStruct((seq, nrow // 2, 2 * d), jnp.float32),
        scratch_types=[
            pltpu.VMEM((m, seq), jnp.int32),
            pltpu.VMEM((_NBUF, seq, d), jnp.float32),
            pltpu.SemaphoreType.DMA,
        ],
        compiler_params=pltpu.CompilerParams(use_tc_tiling_on_sc=False),
    )
    return run(image, table)


def _proj_body(f_ref, w_ref, b_ref, o_ref):
    h = pl.program_id(2)
    f = f_ref[0]
    w = w_ref[...]
    oa = lax.dot_general(
        w, f[:, 0:64], (((1,), (1,)), ((), ())),
        preferred_element_type=jnp.float32,
    )
    ob = lax.dot_general(
        w, f[:, 64:128], (((1,), (1,)), ((), ())),
        preferred_element_type=jnp.float32,
    )
    o_ref[0] = jnp.where(h == 0, oa, ob) + b_ref[...]


def _tc_project(f5, w, bcol):
    seq, half, dd = f5.shape
    d = dd // 2
    nj = half // _NBLK
    return pl.pallas_call(
        _proj_body,
        grid=(seq, nj, 2),
        in_specs=[
            pl.BlockSpec((1, _NBLK, dd), lambda l, jm, h: (l, jm, 0)),
            pl.BlockSpec((d, d), lambda l, jm, h: (0, 0)),
            pl.BlockSpec((d, 1), lambda l, jm, h: (0, 0)),
        ],
        out_specs=pl.BlockSpec((1, d, _NBLK), lambda l, jm, h: (l, 0, h * nj + jm)),
        out_shape=jax.ShapeDtypeStruct((seq, d, 2 * half), jnp.float32),
    )(f5, w, bcol)


def kernel(image, table, W, b):
    v, d = table.shape
    nrow, seq = image.shape

    q2 = _tc_pack_table(table.T)
    q = q2.reshape(2 * q2.shape[0], d)

    im = image.astype(jnp.int32)
    t = im % (2 * _K)
    im2 = 2 * ((im // (2 * _K)) * _K + t % _K) + t // _K

    f5 = _sc_gather(im2, q)
    out_t = _tc_project(f5, W, b.reshape(d, 1))
    return jnp.transpose(out_t, (2, 0, 1))


# speedup vs baseline: 3.4727x; 2.2072x over previous
import functools

import jax
import jax.numpy as jnp
from jax import lax
from jax.experimental import pallas as pl
from jax.experimental.pallas import tpu as pltpu
from jax.experimental.pallas import tpu_sc as plsc

_NC = 2
_NS = 16
_NW = _NC * _NS
_NBUF = 4
_K = 12800
_NBLK = 512


def _pack_body(t_ref, i_ref, o_ref):
    eye = i_ref[...]
    a = lax.dot_general(
        t_ref[:, :_K], eye, (((0,), (0,)), ((), ())),
        preferred_element_type=jnp.float32,
    )
    bb = lax.dot_general(
        t_ref[:, _K:], eye, (((0,), (0,)), ((), ())),
        preferred_element_type=jnp.float32,
    )
    o_ref[:, 0:64] = a
    o_ref[:, 64:128] = bb


def _tc_pack_table(table_t):
    d, v = table_t.shape
    nblk = pl.cdiv(v, 2 * _K)
    return pl.pallas_call(
        _pack_body,
        grid=(nblk,),
        in_specs=[
            pl.BlockSpec((d, 2 * _K), lambda j: (0, j)),
            pl.BlockSpec((d, d), lambda j: (0, 0)),
        ],
        out_specs=pl.BlockSpec((_K, 2 * d), lambda j: (j, 0)),
        out_shape=jax.ShapeDtypeStruct((nblk * _K, 2 * d), jnp.float32),
    )(table_t, jnp.eye(d, dtype=jnp.float32))


def _gather_body(img_hbm, table_hbm, out_hbm, idx_v, rows_v, sem):
    nrow = img_hbm.shape[0]
    m = nrow // _NW
    wid = lax.axis_index("s") * _NC + lax.axis_index("c")
    r0 = wid * m
    m0 = (wid // 2) * m
    off = (wid % 2) * 64

    pltpu.sync_copy(img_hbm.at[pl.ds(r0, m)], idx_v)

    def _start(g, b):
        pltpu.async_copy(table_hbm.at[idx_v.at[g]], rows_v.at[b], sem)

    def _wait_one(b):
        pltpu.make_async_copy(table_hbm.at[idx_v.at[0]], rows_v.at[b], sem).wait()

    for b in range(_NBUF):
        _start(b, b)

    def _group(i, _):
        go = i * _NBUF
        for b in range(_NBUF):
            g = go + b
            _wait_one(b)
            pltpu.sync_copy(rows_v.at[b], out_hbm.at[:, m0 + g, pl.ds(off, 64)])
            nxt = g + _NBUF

            @pl.when(nxt < m)
            def _():
                _start(nxt, b)

        return 0

    lax.fori_loop(0, m // _NBUF, _group, 0)


def _sc_gather(image, table):
    nrow, seq = image.shape
    v, d = table.shape
    m = nrow // _NW
    run = pl.kernel(
        _gather_body,
        mesh=plsc.VectorSubcoreMesh(core_axis_name="c", subcore_axis_name="s"),
        out_type=jax.ShapeDtypeStruct((seq, nrow // 2, 2 * d), jnp.float32),
        scratch_types=[
            pltpu.VMEM((m, seq), jnp.int32),
            pltpu.VMEM((_NBUF, seq, d), jnp.float32),
            pltpu.SemaphoreType.DMA,
        ],
        compiler_params=pltpu.CompilerParams(use_tc_tiling_on_sc=False),
    )
    return run(image, table)


def _proj_body(f_ref, w_ref, b_ref, o_ref):
    f = f_ref[0]
    w = w_ref[...]
    bias = b_ref[...]
    oa = lax.dot_general(
        w, f[:, 0:64], (((1,), (1,)), ((), ())),
        preferred_element_type=jnp.float32,
    ) + bias
    ob = lax.dot_general(
        w, f[:, 64:128], (((1,), (1,)), ((), ())),
        preferred_element_type=jnp.float32,
    ) + bias
    jb = f.shape[0]
    for p in range(jb // _NBLK):
        lo = _NBLK * p
        o_ref[0, :, 2 * lo:2 * lo + _NBLK] = oa[:, lo:lo + _NBLK]
        o_ref[0, :, 2 * lo + _NBLK:2 * lo + 2 * _NBLK] = ob[:, lo:lo + _NBLK]


def _tc_project(f5, w, bcol):
    seq, half, dd = f5.shape
    d = dd // 2
    jb = 2048
    return pl.pallas_call(
        _proj_body,
        grid=(seq, half // jb),
        in_specs=[
            pl.BlockSpec((1, jb, dd), lambda l, jm: (l, jm, 0)),
            pl.BlockSpec((d, d), lambda l, jm: (0, 0)),
            pl.BlockSpec((d, 1), lambda l, jm: (0, 0)),
        ],
        out_specs=pl.BlockSpec((1, d, 2 * jb), lambda l, jm: (l, 0, jm)),
        out_shape=jax.ShapeDtypeStruct((seq, d, 2 * half), jnp.float32),
    )(f5, w, bcol)


def kernel(image, table, W, b):
    v, d = table.shape
    nrow, seq = image.shape

    q2 = _tc_pack_table(table.T)
    q = q2.reshape(2 * q2.shape[0], d)

    im = image.astype(jnp.int32)
    t = im % (2 * _K)
    im2 = 2 * ((im // (2 * _K)) * _K + t % _K) + t // _K

    f5 = _sc_gather(im2, q)
    out_t = _tc_project(f5, W, b.reshape(d, 1))
    return jnp.transpose(out_t, (2, 0, 1))


# speedup vs baseline: 3.9550x; 1.1389x over previous
import functools

import jax
import jax.numpy as jnp
from jax import lax
from jax.experimental import pallas as pl
from jax.experimental.pallas import tpu as pltpu
from jax.experimental.pallas import tpu_sc as plsc

_NC = 2
_NS = 16
_NW = _NC * _NS
_NBUF = 8
_K = 12800
_NBLK = 512


def _pack_body(t_ref, i_ref, o_ref):
    eye = i_ref[...]
    a = lax.dot_general(
        t_ref[:, :_K], eye, (((0,), (0,)), ((), ())),
        preferred_element_type=jnp.float32,
    )
    bb = lax.dot_general(
        t_ref[:, _K:], eye, (((0,), (0,)), ((), ())),
        preferred_element_type=jnp.float32,
    )
    o_ref[:, 0:64] = a
    o_ref[:, 64:128] = bb


def _tc_pack_table(table_t):
    d, v = table_t.shape
    nblk = pl.cdiv(v, 2 * _K)
    return pl.pallas_call(
        _pack_body,
        grid=(nblk,),
        in_specs=[
            pl.BlockSpec((d, 2 * _K), lambda j: (0, j)),
            pl.BlockSpec((d, d), lambda j: (0, 0)),
        ],
        out_specs=pl.BlockSpec((_K, 2 * d), lambda j: (j, 0)),
        out_shape=jax.ShapeDtypeStruct((nblk * _K, 2 * d), jnp.float32),
    )(table_t, jnp.eye(d, dtype=jnp.float32))


def _gather_body(img_hbm, table_hbm, out_hbm, idx_v, rows_v, sem):
    nrow = img_hbm.shape[0]
    m = nrow // _NW
    wid = lax.axis_index("s") * _NC + lax.axis_index("c")
    r0 = wid * m
    m0 = (wid // 2) * m
    off = (wid % 2) * 64

    pltpu.sync_copy(img_hbm.at[pl.ds(r0, m)], idx_v)

    def _start(g, b):
        pltpu.async_copy(table_hbm.at[idx_v.at[g]], rows_v.at[b], sem)

    def _wait_one(b):
        pltpu.make_async_copy(table_hbm.at[idx_v.at[0]], rows_v.at[b], sem).wait()

    for b in range(_NBUF):
        _start(b, b)

    def _group(i, _):
        go = i * _NBUF
        for b in range(_NBUF):
            g = go + b
            _wait_one(b)
            pltpu.sync_copy(rows_v.at[b], out_hbm.at[:, m0 + g, pl.ds(off, 64)])
            nxt = g + _NBUF

            @pl.when(nxt < m)
            def _():
                _start(nxt, b)

        return 0

    lax.fori_loop(0, m // _NBUF, _group, 0)


def _sc_gather(image, table):
    nrow, seq = image.shape
    v, d = table.shape
    m = nrow // _NW
    run = pl.kernel(
        _gather_body,
        mesh=plsc.VectorSubcoreMesh(core_axis_name="c", subcore_axis_name="s"),
        out_type=jax.ShapeDtypeStruct((seq, nrow // 2, 2 * d), jnp.float32),
        scratch_types=[
            pltpu.VMEM((m, seq), jnp.int32),
            pltpu.VMEM((_NBUF, seq, d), jnp.float32),
            pltpu.SemaphoreType.DMA,
        ],
        compiler_params=pltpu.CompilerParams(use_tc_tiling_on_sc=False),
    )
    return run(image, table)


def _proj_body(f_ref, w_ref, b_ref, o_ref):
    f = f_ref[0]
    w = w_ref[...]
    bias = b_ref[...]
    oa = lax.dot_general(
        w, f[:, 0:64], (((1,), (1,)), ((), ())),
        preferred_element_type=jnp.float32,
    ) + bias
    ob = lax.dot_general(
        w, f[:, 64:128], (((1,), (1,)), ((), ())),
        preferred_element_type=jnp.float32,
    ) + bias
    jb = f.shape[0]
    for p in range(jb // _NBLK):
        lo = _NBLK * p
        o_ref[0, :, 2 * lo:2 * lo + _NBLK] = oa[:, lo:lo + _NBLK]
        o_ref[0, :, 2 * lo + _NBLK:2 * lo + 2 * _NBLK] = ob[:, lo:lo + _NBLK]


def _tc_project(f5, w, bcol):
    seq, half, dd = f5.shape
    d = dd // 2
    jb = 4096
    return pl.pallas_call(
        _proj_body,
        grid=(seq, half // jb),
        in_specs=[
            pl.BlockSpec((1, jb, dd), lambda l, jm: (l, jm, 0)),
            pl.BlockSpec((d, d), lambda l, jm: (0, 0)),
            pl.BlockSpec((d, 1), lambda l, jm: (0, 0)),
        ],
        out_specs=pl.BlockSpec((1, d, 2 * jb), lambda l, jm: (l, 0, jm)),
        out_shape=jax.ShapeDtypeStruct((seq, d, 2 * half), jnp.float32),
    )(f5, w, bcol)


def kernel(image, table, W, b):
    v, d = table.shape
    nrow, seq = image.shape

    q2 = _tc_pack_table(table.T)
    q = q2.reshape(2 * q2.shape[0], d)

    im = image.astype(jnp.int32)
    t = im % (2 * _K)
    im2 = 2 * ((im // (2 * _K)) * _K + t % _K) + t // _K

    f5 = _sc_gather(im2, q)
    out_t = _tc_project(f5, W, b.reshape(d, 1))
    return jnp.transpose(out_t, (2, 0, 1))


# speedup vs baseline: 4.1374x; 1.0461x over previous
import functools

import jax
import jax.numpy as jnp
from jax import lax
from jax.experimental import pallas as pl
from jax.experimental.pallas import tpu as pltpu
from jax.experimental.pallas import tpu_sc as plsc

_NC = 2
_NS = 16
_NW = _NC * _NS
_NBUF = 8
_K = 12800
_NBLK = 512


def _pack_body(t_ref, i_ref, o_ref):
    eye = i_ref[...]
    a = lax.dot_general(
        t_ref[:, :_K], eye, (((0,), (0,)), ((), ())),
        preferred_element_type=jnp.float32,
    )
    bb = lax.dot_general(
        t_ref[:, _K:], eye, (((0,), (0,)), ((), ())),
        preferred_element_type=jnp.float32,
    )
    o_ref[:, 0:64] = a
    o_ref[:, 64:128] = bb


def _tc_pack_table(table_t):
    d, v = table_t.shape
    nblk = pl.cdiv(v, 2 * _K)
    return pl.pallas_call(
        _pack_body,
        grid=(nblk,),
        in_specs=[
            pl.BlockSpec((d, 2 * _K), lambda j: (0, j)),
            pl.BlockSpec((d, d), lambda j: (0, 0)),
        ],
        out_specs=pl.BlockSpec((_K, 2 * d), lambda j: (j, 0)),
        out_shape=jax.ShapeDtypeStruct((nblk * _K, 2 * d), jnp.float32),
    )(table_t, jnp.eye(d, dtype=jnp.float32))


_CH = 128


def _gather_body(imt_hbm, table_hbm, out_hbm, idx_v, rows_v, sem):
    seq, nrow = imt_hbm.shape
    m = nrow // _NW
    nch = m // _CH
    wid = lax.axis_index("s") * _NC + lax.axis_index("c")
    r0 = wid * m
    m0 = (wid // 2) * m
    off = (wid % 2) * 64

    pltpu.sync_copy(imt_hbm.at[:, pl.ds(r0, m)], idx_v)

    def _start(q, b):
        l, c0 = q // nch, (q % nch) * _CH
        pltpu.async_copy(
            table_hbm.at[idx_v.at[l, pl.ds(c0, _CH)]], rows_v.at[b], sem
        )

    def _wait_one(b):
        pltpu.make_async_copy(
            table_hbm.at[idx_v.at[0, pl.ds(0, _CH)]], rows_v.at[b], sem
        ).wait()

    for b in range(_NBUF):
        _start(b, b)

    nq = seq * nch

    def _group(i, _):
        go = i * _NBUF
        for b in range(_NBUF):
            q = go + b
            l, c0 = q // nch, (q % nch) * _CH
            _wait_one(b)
            pltpu.sync_copy(
                rows_v.at[b], out_hbm.at[l, pl.ds(m0 + c0, _CH), pl.ds(off, 64)]
            )
            nxt = q + _NBUF

            @pl.when(nxt < nq)
            def _():
                _start(nxt, b)

        return 0

    lax.fori_loop(0, nq // _NBUF, _group, 0)


def _sc_gather(imaget, table):
    seq, nrow = imaget.shape
    v, d = table.shape
    m = nrow // _NW
    run = pl.kernel(
        _gather_body,
        mesh=plsc.VectorSubcoreMesh(core_axis_name="c", subcore_axis_name="s"),
        out_type=jax.ShapeDtypeStruct((seq, nrow // 2, 2 * d), jnp.float32),
        scratch_types=[
            pltpu.VMEM((seq, m), jnp.int32),
            pltpu.VMEM((_NBUF, _CH, d), jnp.float32),
            pltpu.SemaphoreType.DMA,
        ],
        compiler_params=pltpu.CompilerParams(use_tc_tiling_on_sc=False),
    )
    return run(imaget, table)


def _proj_body(f_ref, w_ref, b_ref, o_ref):
    f = f_ref[0]
    w = w_ref[...]
    bias = b_ref[...]
    oa = lax.dot_general(
        w, f[:, 0:64], (((1,), (1,)), ((), ())),
        preferred_element_type=jnp.float32,
    ) + bias
    ob = lax.dot_general(
        w, f[:, 64:128], (((1,), (1,)), ((), ())),
        preferred_element_type=jnp.float32,
    ) + bias
    jb = f.shape[0]
    for p in range(jb // _NBLK):
        lo = _NBLK * p
        o_ref[0, :, 2 * lo:2 * lo + _NBLK] = oa[:, lo:lo + _NBLK]
        o_ref[0, :, 2 * lo + _NBLK:2 * lo + 2 * _NBLK] = ob[:, lo:lo + _NBLK]


def _tc_project(f5, w, bcol):
    seq, half, dd = f5.shape
    d = dd // 2
    jb = 4096
    return pl.pallas_call(
        _proj_body,
        grid=(seq, half // jb),
        in_specs=[
            pl.BlockSpec((1, jb, dd), lambda l, jm: (l, jm, 0)),
            pl.BlockSpec((d, d), lambda l, jm: (0, 0)),
            pl.BlockSpec((d, 1), lambda l, jm: (0, 0)),
        ],
        out_specs=pl.BlockSpec((1, d, 2 * jb), lambda l, jm: (l, 0, jm)),
        out_shape=jax.ShapeDtypeStruct((seq, d, 2 * half), jnp.float32),
    )(f5, w, bcol)


def kernel(image, table, W, b):
    v, d = table.shape
    nrow, seq = image.shape

    q2 = _tc_pack_table(table.T)
    q = q2.reshape(2 * q2.shape[0], d)

    im = image.T.astype(jnp.int32)
    t = im % (2 * _K)
    im2 = 2 * ((im // (2 * _K)) * _K + t % _K) + t // _K

    f5 = _sc_gather(im2, q)
    out_t = _tc_project(f5, W, b.reshape(d, 1))
    return jnp.transpose(out_t, (2, 0, 1))
